# Initial kernel scaffold; baseline (speedup 1.0000x reference)
import jax
import jax.numpy as jnp
from jax.experimental import pallas as pl


def kernel(x, edge_index, depth, batch, emb, dpW, dpB, lin0, asrc0, adst0, bias0, bnw0, bnb0, lin1, asrc1, adst1, bias1, bnw1, bnb1, lin2, asrc2, adst2, bias2, bnw2, bnb2, cW1, cb1, cW2, cb2):
    raise NotImplementedError("write your pallas kernel here")


# speedup vs baseline: 1.1204x; 1.1204x over previous
import jax
import jax.numpy as jnp
from jax.experimental import pallas as pl

_N = 50000
_HEADS = 4
_HID = 32
_G2 = 64
_NUM_GRAPHS = 64
_NUM_CLASSES = 20


def _gat_layer(h, src, dst, W, a_s, a_d, b, heads, dout, concat):
    n = h.shape[0]
    hw = (h @ W).reshape(n, heads, dout)
    al_s = (hw * a_s[None, :, :]).sum(-1)
    al_d = (hw * a_d[None, :, :]).sum(-1)
    e = al_s[src] + al_d[dst]
    e = jnp.where(e > 0, e, 0.2 * e)
    ex = jnp.exp(e)
    den = jax.ops.segment_sum(ex, dst, num_segments=n)
    out = jax.ops.segment_sum(hw[src] * ex[:, :, None], dst, num_segments=n)
    out = out / (den[:, :, None] + 1e-16)
    if concat:
        out = out.reshape(n, heads * dout)
    else:
        out = out.mean(axis=1)
    return out + b


def _head_kernel(g_ref, w1_ref, b1_ref, w2_ref, b2_ref, o_ref):
    g = g_ref[...]
    hid = jnp.maximum(g @ w1_ref[...] + b1_ref[...][None, :], 0.0)
    o_ref[...] = hid @ w2_ref[...] + b2_ref[...][None, :]


def kernel(x, edge_index, depth, batch, emb, dpW, dpB, lin0, asrc0, adst0, bias0, bnw0, bnb0, lin1, asrc1, adst1, bias1, bnw1, bnb1, lin2, asrc2, adst2, bias2, bnw2, bnb2, cW1, cb1, cW2, cb2):
    loop = jnp.arange(_N)
    src = jnp.concatenate([edge_index[0], loop])
    dst = jnp.concatenate([edge_index[1], loop])
    inv = 1.0 / jnp.sqrt(1.0 + 1e-5)
    h = emb[x] + depth[:, None] @ dpW + dpB
    h = _gat_layer(h, src, dst, lin0, asrc0, adst0, bias0, _HEADS, _HID, True)
    h = jax.nn.elu(h * (bnw0 * inv) + bnb0)
    h = _gat_layer(h, src, dst, lin1, asrc1, adst1, bias1, _HEADS, _HID, True)
    h = jax.nn.elu(h * (bnw1 * inv) + bnb1)
    h = _gat_layer(h, src, dst, lin2, asrc2, adst2, bias2, 1, _G2, False)
    h = jax.nn.elu(h * (bnw2 * inv) + bnb2)
    cnt = jax.ops.segment_sum(jnp.ones((_N,), dtype=h.dtype), batch, num_segments=_NUM_GRAPHS)
    hmean = jax.ops.segment_sum(h, batch, num_segments=_NUM_GRAPHS) / jnp.maximum(cnt, 1.0)[:, None]
    hmax = jax.ops.segment_max(h, batch, num_segments=_NUM_GRAPHS)
    g = jnp.concatenate([hmean, hmax], axis=-1)
    out = pl.pallas_call(
        _head_kernel,
        out_shape=jax.ShapeDtypeStruct((_NUM_GRAPHS, _NUM_CLASSES), jnp.float32),
    )(g, cW1, cb1, cW2, cb2)
    return out


# speedup vs baseline: 29.7128x; 26.5199x over previous
import dataclasses
import functools

import jax
import jax.numpy as jnp
from jax import lax
from jax.experimental import pallas as pl
from jax.experimental.pallas import tpu as pltpu
from jax.experimental.pallas import tpu_sc as plsc

_N = 50000
_E = 800000
_EN = _E + _N
_NUM_TYPES = 512
_NUM_GRAPHS = 64
_D_EMB = 64
_HID = 32
_HEADS = 4
_G2 = 64
_NUM_CLASSES = 20

_R = 512
_NPAD = 50176
_NBLK = _NPAD // _R
_CS = _NPAD // 8
_K = 128
_EPAD = 860160
_INV = float(1.0 / (1.0 + 1e-5) ** 0.5)
_TW0 = 256
_TW2 = 128
_ROW0 = 144
_ROW2 = 80


def _make_edge_kernel(GROW, AROW, H, D):
    DH = D // H
    CR = 224
    NCH = 7

    mesh = plsc.VectorSubcoreMesh(
        core_axis_name="c", subcore_axis_name="s", num_cores=2, num_subcores=16
    )
    cp = pltpu.CompilerParams()
    if "needs_layout_passes" in pltpu.CompilerParams.__dataclass_fields__:
        cp = dataclasses.replace(cp, needs_layout_passes=False)

    @functools.partial(
        pl.kernel,
        out_type=jax.ShapeDtypeStruct((_NPAD, AROW), jnp.float32),
        mesh=mesh,
        compiler_params=cp,
        scratch_types=[
            pltpu.VMEM((_K,), jnp.int32),
            pltpu.VMEM((_K,), jnp.int32),
            pltpu.VMEM((_K,), jnp.int32),
            pltpu.VMEM((_K, GROW), jnp.float32),
            pltpu.VMEM((CR, 4), jnp.float32),
            pltpu.VMEM((240,), jnp.int32),
            pltpu.VMEM((CR + 8, AROW), jnp.float32),
            pltpu.SemaphoreType.DMA,
            pltpu.SemaphoreType.DMA,
        ],
    )
    def edge_kernel(t_hbm, ald_hbm, src_hbm, dst_hbm, bounds_hbm, s_hbm,
                    srcbuf, dstbuf, dstloc, rows, aldv, bvec, acc,
                    sem, sem2):
        core = lax.axis_index("c")
        sid = lax.axis_index("s")
        w = core * 16 + sid
        pltpu.sync_copy(bounds_hbm, bvec)

        def bscal(idx):
            v = plsc.load_gather(bvec, [jnp.full((16,), idx, jnp.int32)])
            return jnp.max(v)

        zf = jnp.zeros((16,), jnp.float32)

        for i in range(NCH):
            c = w * NCH + i
            cbase = c * CR
            lo = bscal(c)
            hi = bscal(c + 1)

            @pl.loop(0, CR + 8)
            def _(r):
                for q in range(AROW // 16):
                    acc[r, pl.ds(q * 16, 16)] = zf

            pltpu.sync_copy(ald_hbm.at[pl.ds(cbase, CR)], aldv)

            lo_al = (lo // 8) * 8
            nb = (hi - lo_al + (_K - 1)) // _K

            @pl.loop(0, nb)
            def _(b):
                base = lo_al + b * _K
                cp1 = pltpu.async_copy(src_hbm.at[pl.ds(base, _K)], srcbuf, sem)
                cp2 = pltpu.async_copy(dst_hbm.at[pl.ds(base, _K)], dstbuf, sem)
                cp1.wait()
                cp2.wait()
                pltpu.async_copy(t_hbm.at[srcbuf], rows, sem2).wait()

                for g in range(_K // 16):
                    ids = lax.iota(jnp.int32, 16) + (g * 16)
                    gid = ids + base
                    mask = (gid >= lo) & (gid < hi)
                    dv = dstbuf[pl.ds(g * 16, 16)] - cbase
                    dvc = jnp.minimum(jnp.maximum(dv, 0), CR - 1)
                    dstloc[pl.ds(g * 16, 16)] = jnp.where(mask, dvc, CR)
                    for h in range(H):
                        colv = jnp.full((16,), D + h, jnp.int32)
                        als = plsc.load_gather(rows, [ids, colv])
                        ald = plsc.load_gather(
                            aldv, [dvc, jnp.full((16,), h, jnp.int32)])
                        e = als + ald
                        e = jnp.where(e > 0, e, 0.2 * e)
                        ex = jnp.exp(e)
                        plsc.store_scatter(rows, [ids, colv], ex)

                @pl.loop(0, _K)
                def _(j):
                    jv = jnp.full((16,), j, jnp.int32)
                    dvj = jnp.max(plsc.load_gather(dstloc, [jv]))
                    for h in range(H):
                        bc = plsc.load_gather(
                            rows, [jv, jnp.full((16,), D + h, jnp.int32)])
                        for q in range(DH // 16):
                            sl = pl.ds(h * DH + q * 16, 16)
                            plsc.addupdate(acc.at[dvj, sl], rows[j, sl] * bc)
                    plsc.addupdate(acc.at[dvj, pl.ds(D, 16)],
                                   rows[j, pl.ds(D, 16)])

            pltpu.sync_copy(acc.at[pl.ds(0, CR)],
                            s_hbm.at[pl.ds(cbase, CR)])

    return edge_kernel


_edge_kernel_l01 = _make_edge_kernel(_TW0, _ROW0, _HEADS, _HEADS * _HID)
_edge_kernel_l2 = _make_edge_kernel(_TW2, _ROW2, 1, _G2)


_HP = functools.partial(jnp.dot, precision=lax.Precision.HIGHEST)


def _k0_body(x_ref, dep_ref, emb_ref, dpw_ref, dpb_ref, lin_ref,
             as_ref, ad_ref, t_ref, ald_ref):
    ids = x_ref[0, 0, :]
    oh = (ids[:, None] ==
          lax.broadcasted_iota(jnp.int32, (_R, _NUM_TYPES), 1)
          ).astype(jnp.float32)
    h = _HP(oh, emb_ref[...])
    dep = dep_ref[0, 0, :]
    h = h + dep[:, None] * dpw_ref[...] + dpb_ref[...]
    hw = _HP(h, lin_ref[...])
    als = _HP(hw, as_ref[...])
    ald = _HP(hw, ad_ref[...])
    pad = jnp.zeros((_R, _TW0 - als.shape[1] - hw.shape[1]), jnp.float32)
    t_ref[...] = jnp.concatenate([hw, als, pad], axis=1)
    if ald.shape[1] < 4:
        ald = jnp.concatenate(
            [ald, jnp.zeros((_R, 4 - ald.shape[1]), jnp.float32)], axis=1)
    ald_ref[...] = ald


def _mid_body(s_ref, bias_ref, bnw_ref, bnb_ref, lin_ref, as_ref, ad_ref,
              t_ref, ald_ref, *, Dp, Hp, ROWn):
    DHp = Dp // Hp
    s = s_ref[...]
    outs = []
    for h in range(Hp):
        num = s[:, h * DHp:(h + 1) * DHp]
        den = s[:, Dp + h:Dp + h + 1]
        outs.append(num / (den + 1e-16))
    out = jnp.concatenate(outs, axis=1) if Hp > 1 else outs[0]
    out = out + bias_ref[...]
    hh = out * (bnw_ref[...] * _INV) + bnb_ref[...]
    hact = jnp.where(hh > 0, hh, jnp.exp(hh) - 1.0)
    hw = _HP(hact, lin_ref[...])
    als = _HP(hw, as_ref[...])
    ald = _HP(hw, ad_ref[...])
    pad = jnp.zeros((_R, ROWn - als.shape[1] - hw.shape[1]), jnp.float32)
    t_ref[...] = jnp.concatenate([hw, als, pad], axis=1)
    if ald.shape[1] < 4:
        ald = jnp.concatenate(
            [ald, jnp.zeros((_R, 4 - ald.shape[1]), jnp.float32)], axis=1)
    ald_ref[...] = ald


def _final_body(s_ref, bias_ref, bnw_ref, bnb_ref, b_ref,
                w1_ref, b1_ref, w2_ref, b2_ref, o_ref, accs, accm):
    i = pl.program_id(0)

    @pl.when(i == 0)
    def _():
        accs[...] = jnp.zeros_like(accs)
        accm[...] = jnp.full_like(accm, -1e30)

    s = s_ref[...]
    num = s[:, :_G2]
    den = s[:, _G2:_G2 + 1]
    out = num / (den + 1e-16) + bias_ref[...]
    hh = out * (bnw_ref[...] * _INV) + bnb_ref[...]
    hact = jnp.where(hh > 0, hh, jnp.exp(hh) - 1.0)

    b = b_ref[0, 0, :]
    rid = lax.broadcasted_iota(jnp.int32, (_R, 1), 0) + i * _R
    valid = rid < _N
    ohT = ((b[None, :] ==
            lax.broadcasted_iota(jnp.int32, (_NUM_GRAPHS, _R), 0))
           & (valid[:, 0])[None, :]).astype(jnp.float32)
    hcnt = jnp.concatenate(
        [hact, jnp.ones((_R, 1), jnp.float32),
         jnp.zeros((_R, 2 * _G2 - _G2 - 1), jnp.float32)], axis=1)
    accs[...] += _HP(ohT, hcnt)

    for g in range(_NUM_GRAPHS):
        m = (b[:, None] == g) & valid
        hm = jnp.where(m, hact, -1e30)
        accm[g, :] = jnp.maximum(accm[g, :], jnp.max(hm, axis=0))

    @pl.when(i == _NBLK - 1)
    def _():
        sums = accs[:, :_G2]
        cnt = accs[:, _G2:_G2 + 1]
        hmean = sums / jnp.maximum(cnt, 1.0)
        gfeat = jnp.concatenate([hmean, accm[...]], axis=1)
        hid = jnp.maximum(_HP(gfeat, w1_ref[...]) + b1_ref[...], 0.0)
        o_ref[...] = _HP(hid, w2_ref[...]) + b2_ref[...]


def _full(shape):
    return pl.BlockSpec(shape, lambda i: (0,) * len(shape))


def _k0(x3, dep3, emb, dpw, dpb, lin, As, Ad):
    return pl.pallas_call(
        _k0_body,
        grid=(_NBLK,),
        in_specs=[
            pl.BlockSpec((1, 1, _R), lambda i: (i, 0, 0)),
            pl.BlockSpec((1, 1, _R), lambda i: (i, 0, 0)),
            _full(emb.shape), _full(dpw.shape), _full(dpb.shape),
            _full(lin.shape), _full(As.shape), _full(Ad.shape),
        ],
        out_specs=[
            pl.BlockSpec((_R, _TW0), lambda i: (i, 0)),
            pl.BlockSpec((_R, 4), lambda i: (i, 0)),
        ],
        out_shape=[
            jax.ShapeDtypeStruct((_NPAD, _TW0), jnp.float32),
            jax.ShapeDtypeStruct((_NPAD, 4), jnp.float32),
        ],
    )(x3, dep3, emb, dpw, dpb, lin, As, Ad)


def _kmid(S, bias, bnw, bnb, lin, As, Ad, Dp, Hp, ROWn):
    body = functools.partial(_mid_body, Dp=Dp, Hp=Hp, ROWn=ROWn)
    return pl.pallas_call(
        body,
        grid=(_NBLK,),
        in_specs=[
            pl.BlockSpec((_R, S.shape[1]), lambda i: (i, 0)),
            _full(bias.shape), _full(bnw.shape), _full(bnb.shape),
            _full(lin.shape), _full(As.shape), _full(Ad.shape),
        ],
        out_specs=[
            pl.BlockSpec((_R, ROWn), lambda i: (i, 0)),
            pl.BlockSpec((_R, 4), lambda i: (i, 0)),
        ],
        out_shape=[
            jax.ShapeDtypeStruct((_NPAD, ROWn), jnp.float32),
            jax.ShapeDtypeStruct((_NPAD, 4), jnp.float32),
        ],
    )(S, bias, bnw, bnb, lin, As, Ad)


def _kfinal(S, bias, bnw, bnb, b3, w1, b1, w2, b2):
    return pl.pallas_call(
        _final_body,
        grid=(_NBLK,),
        in_specs=[
            pl.BlockSpec((_R, _ROW2), lambda i: (i, 0)),
            _full(bias.shape), _full(bnw.shape), _full(bnb.shape),
            pl.BlockSpec((1, 1, _R), lambda i: (i, 0, 0)),
            _full(w1.shape), _full(b1.shape), _full(w2.shape),
            _full(b2.shape),
        ],
        out_specs=pl.BlockSpec((_NUM_GRAPHS, _NUM_CLASSES), lambda i: (0, 0)),
        out_shape=jax.ShapeDtypeStruct((_NUM_GRAPHS, _NUM_CLASSES),
                                       jnp.float32),
        scratch_shapes=[
            pltpu.VMEM((_NUM_GRAPHS, 2 * _G2), jnp.float32),
            pltpu.VMEM((_NUM_GRAPHS, _G2), jnp.float32),
        ],
    )(S, bias, bnw, bnb, b3, w1, b1, w2, b2)


def _blockdiag(a, D):
    H, DH = a.shape
    cols = []
    for h in range(H):
        col = jnp.zeros((D, 1), jnp.float32)
        col = lax.dynamic_update_slice(col, a[h][:, None], (h * DH, 0))
        cols.append(col)
    return jnp.concatenate(cols, axis=1)


def _pad_reshape_3d(v):
    v = jnp.concatenate([v, jnp.zeros((_NPAD - _N,), v.dtype)])
    return v.reshape(_NBLK, 1, _R)


def kernel(x, edge_index, depth, batch, emb, dpW, dpB, lin0, asrc0, adst0, bias0, bnw0, bnb0, lin1, asrc1, adst1, bias1, bnw1, bnb1, lin2, asrc2, adst2, bias2, bnw2, bnb2, cW1, cb1, cW2, cb2):
    loop = jnp.arange(_N, dtype=jnp.int32)
    src = jnp.concatenate([edge_index[0].astype(jnp.int32), loop])
    dst = jnp.concatenate([edge_index[1].astype(jnp.int32), loop])
    dsts, srcs = lax.sort((dst, src), num_keys=1)
    zpad = jnp.zeros((_EPAD - _EN,), jnp.int32)
    srcs = jnp.concatenate([srcs, zpad])
    dsts = jnp.concatenate([dsts, zpad])
    cuts = jnp.arange(225, dtype=jnp.int32) * 224
    lob = jnp.searchsorted(dsts[:_EN], cuts[:224]).astype(jnp.int32)
    bounds = jnp.concatenate(
        [lob, jnp.array([_EN], jnp.int32), jnp.zeros((15,), jnp.int32)])

    x3 = _pad_reshape_3d(x.astype(jnp.int32))
    dep3 = _pad_reshape_3d(depth)
    b3 = _pad_reshape_3d(batch.astype(jnp.int32))

    r1 = lambda v: v.reshape(1, -1)
    T0, ALD0 = _k0(x3, dep3, emb, dpW, r1(dpB), lin0,
                   _blockdiag(asrc0, _HEADS * _HID),
                   _blockdiag(adst0, _HEADS * _HID))
    S0 = _edge_kernel_l01(T0, ALD0, srcs, dsts, bounds)
    T1, ALD1 = _kmid(S0, r1(bias0), r1(bnw0), r1(bnb0), lin1,
                     _blockdiag(asrc1, _HEADS * _HID),
                     _blockdiag(adst1, _HEADS * _HID),
                     _HEADS * _HID, _HEADS, _TW0)
    S1 = _edge_kernel_l01(T1, ALD1, srcs, dsts, bounds)
    T2, ALD2 = _kmid(S1, r1(bias1), r1(bnw1), r1(bnb1), lin2,
                     _blockdiag(asrc2, _G2), _blockdiag(adst2, _G2),
                     _HEADS * _HID, _HEADS, _TW2)
    S2 = _edge_kernel_l2(T2, ALD2, srcs, dsts, bounds)
    return _kfinal(S2, r1(bias2), r1(bnw2), r1(bnb2), b3,
                   cW1, r1(cb1), cW2, r1(cb2))


# speedup vs baseline: 32.5118x; 1.0942x over previous
import dataclasses
import functools

import jax
import jax.numpy as jnp
from jax import lax
from jax.experimental import pallas as pl
from jax.experimental.pallas import tpu as pltpu
from jax.experimental.pallas import tpu_sc as plsc

_N = 50000
_E = 800000
_EN = _E + _N
_NUM_TYPES = 512
_NUM_GRAPHS = 64
_D_EMB = 64
_HID = 32
_HEADS = 4
_G2 = 64
_NUM_CLASSES = 20

_R = 512
_NPAD = 50176
_NBLK = _NPAD // _R
_CS = _NPAD // 8
_K = 128
_EPAD = 860160
_INV = float(1.0 / (1.0 + 1e-5) ** 0.5)
_TW0 = 256
_TW2 = 128
_ROW0 = 144
_ROW2 = 80


def _make_edge_kernel(GROW, AROW, H, D):
    DH = D // H
    CR = 224
    NCH = 7

    mesh = plsc.VectorSubcoreMesh(
        core_axis_name="c", subcore_axis_name="s", num_cores=2, num_subcores=16
    )
    cp = pltpu.CompilerParams()
    if "needs_layout_passes" in pltpu.CompilerParams.__dataclass_fields__:
        cp = dataclasses.replace(cp, needs_layout_passes=False)

    @functools.partial(
        pl.kernel,
        out_type=jax.ShapeDtypeStruct((_NPAD * AROW,), jnp.float32),
        mesh=mesh,
        compiler_params=cp,
        scratch_types=[
            pltpu.VMEM((_K,), jnp.int32),
            pltpu.VMEM((_K,), jnp.int32),
            pltpu.VMEM((_K,), jnp.int32),
            pltpu.VMEM((_K,), jnp.int32),
            pltpu.VMEM((_K, GROW), jnp.float32),
            pltpu.VMEM((_K, GROW), jnp.float32),
            pltpu.VMEM((_K,), jnp.int32),
            pltpu.VMEM((CR * 4,), jnp.float32),
            pltpu.VMEM((240,), jnp.int32),
            pltpu.VMEM(((CR + 8) * AROW,), jnp.float32),
            pltpu.SemaphoreType.DMA,
            pltpu.SemaphoreType.DMA,
            pltpu.SemaphoreType.DMA,
            pltpu.SemaphoreType.DMA,
        ],
    )
    def edge_kernel(t_hbm, ald_hbm, src_hbm, dst_hbm, bounds_hbm, s_hbm,
                    srcb0, dstb0, srcb1, dstb1, rows0, rows1, dstloc,
                    aldv, bvec, acc, semi0, semi1, semg0, semg1):
        core = lax.axis_index("c")
        sid = lax.axis_index("s")
        w = core * 16 + sid
        pltpu.sync_copy(bounds_hbm, bvec)

        def bscal(idx):
            v = plsc.load_gather(bvec, [jnp.full((16,), idx, jnp.int32)])
            return jnp.max(v)

        zf = jnp.zeros((16,), jnp.float32)
        sets = ((srcb0, dstb0, rows0, semi0, semg0),
                (srcb1, dstb1, rows1, semi1, semg1))

        @pl.loop(0, NCH)
        def _(i):
            c = w * NCH + i
            cbase = c * CR
            lo = bscal(c)
            hi = bscal(c + 1)

            @pl.loop(0, (CR + 8) * AROW // 16)
            def _(r):
                acc[pl.ds(r * 16, 16)] = zf

            pltpu.sync_copy(ald_hbm.at[pl.ds(cbase * 4, CR * 4)], aldv)

            lo_al = (lo // 8) * 8
            nb = (hi - lo_al + (_K - 1)) // _K
            t2 = (nb + 1) // 2

            def fire_idx(s, b):
                sb, db, _, smi, _ = sets[s]
                base = lo_al + b * _K
                pltpu.async_copy(src_hbm.at[pl.ds(base, _K)], sb, smi)
                pltpu.async_copy(dst_hbm.at[pl.ds(base, _K)], db, smi)

            def wait_idx(s):
                sb, db, _, smi, _ = sets[s]
                pltpu.make_async_copy(
                    src_hbm.at[pl.ds(lo_al, _K)], sb, smi).wait()
                pltpu.make_async_copy(
                    dst_hbm.at[pl.ds(lo_al, _K)], db, smi).wait()

            def fire_g(s):
                sb, _, rw, _, smg = sets[s]
                pltpu.async_copy(t_hbm.at[sb], rw, smg)

            def wait_g(s):
                sb, _, rw, _, smg = sets[s]
                pltpu.make_async_copy(t_hbm.at[sb], rw, smg).wait()

            def compute(s, b):
                _, db, rw, _, _ = sets[s]
                base = lo_al + b * _K
                for g in range(_K // 16):
                    ids = lax.iota(jnp.int32, 16) + (g * 16)
                    gid = ids + base
                    mask = (gid >= lo) & (gid < hi)
                    dv = db[pl.ds(g * 16, 16)] - cbase
                    dvc = jnp.minimum(jnp.maximum(dv, 0), CR - 1)
                    dstloc[pl.ds(g * 16, 16)] = jnp.where(mask, dvc, CR)
                    for h in range(H):
                        colv = jnp.full((16,), D + h, jnp.int32)
                        als = plsc.load_gather(rw, [ids, colv])
                        ald = plsc.load_gather(aldv, [dvc * 4 + h])
                        e = als + ald
                        e = jnp.where(e > 0, e, 0.2 * e)
                        ex = jnp.exp(e)
                        plsc.store_scatter(rw, [ids, colv], ex)

                @pl.loop(0, _K)
                def _(j):
                    jv = jnp.full((16,), j, jnp.int32)
                    dvj = jnp.max(plsc.load_gather(dstloc, [jv]))
                    for h in range(H):
                        bc = plsc.load_gather(
                            rw, [jv, jnp.full((16,), D + h, jnp.int32)])
                        for q in range(DH // 16):
                            co = h * DH + q * 16
                            plsc.addupdate(
                                acc.at[pl.ds(dvj * AROW + co, 16)],
                                rw[j, pl.ds(co, 16)] * bc)
                    plsc.addupdate(acc.at[pl.ds(dvj * AROW + D, 16)],
                                   rw[j, pl.ds(D, 16)])

            fire_idx(0, 0)
            wait_idx(0)
            fire_g(0)
            fire_idx(1, 1)

            @pl.loop(0, t2)
            def _(t):
                b0 = 2 * t
                wait_idx(1)
                fire_g(1)
                wait_g(0)
                compute(0, b0)
                fire_idx(0, b0 + 2)
                wait_idx(0)
                fire_g(0)
                wait_g(1)
                compute(1, b0 + 1)
                fire_idx(1, b0 + 3)

            wait_g(0)
            wait_idx(1)

            pltpu.sync_copy(acc.at[pl.ds(0, CR * AROW)],
                            s_hbm.at[pl.ds(cbase * AROW, CR * AROW)])

    return edge_kernel


_edge_kernel_l01 = _make_edge_kernel(_TW0, _ROW0, _HEADS, _HEADS * _HID)
_edge_kernel_l2 = _make_edge_kernel(_TW2, _ROW2, 1, _G2)


_HP = functools.partial(jnp.dot, precision=lax.Precision.HIGHEST)


def _k0_body(x_ref, dep_ref, emb_ref, dpw_ref, dpb_ref, lin_ref,
             as_ref, ad_ref, t_ref, ald_ref):
    ids = x_ref[0, 0, :]
    oh = (ids[:, None] ==
          lax.broadcasted_iota(jnp.int32, (_R, _NUM_TYPES), 1)
          ).astype(jnp.float32)
    h = _HP(oh, emb_ref[...])
    dep = dep_ref[0, 0, :]
    h = h + dep[:, None] * dpw_ref[...] + dpb_ref[...]
    hw = _HP(h, lin_ref[...])
    als = _HP(hw, as_ref[...])
    ald = _HP(hw, ad_ref[...])
    pad = jnp.zeros((_R, _TW0 - als.shape[1] - hw.shape[1]), jnp.float32)
    t_ref[...] = jnp.concatenate([hw, als, pad], axis=1)
    if ald.shape[1] < 4:
        ald = jnp.concatenate(
            [ald, jnp.zeros((_R, 4 - ald.shape[1]), jnp.float32)], axis=1)
    ald_ref[...] = ald


def _mid_body(s_ref, bias_ref, bnw_ref, bnb_ref, lin_ref, as_ref, ad_ref,
              t_ref, ald_ref, *, Dp, Hp, ROWn):
    DHp = Dp // Hp
    s = s_ref[...]
    outs = []
    for h in range(Hp):
        num = s[:, h * DHp:(h + 1) * DHp]
        den = s[:, Dp + h:Dp + h + 1]
        outs.append(num / (den + 1e-16))
    out = jnp.concatenate(outs, axis=1) if Hp > 1 else outs[0]
    out = out + bias_ref[...]
    hh = out * (bnw_ref[...] * _INV) + bnb_ref[...]
    hact = jnp.where(hh > 0, hh, jnp.exp(hh) - 1.0)
    hw = _HP(hact, lin_ref[...])
    als = _HP(hw, as_ref[...])
    ald = _HP(hw, ad_ref[...])
    pad = jnp.zeros((_R, ROWn - als.shape[1] - hw.shape[1]), jnp.float32)
    t_ref[...] = jnp.concatenate([hw, als, pad], axis=1)
    if ald.shape[1] < 4:
        ald = jnp.concatenate(
            [ald, jnp.zeros((_R, 4 - ald.shape[1]), jnp.float32)], axis=1)
    ald_ref[...] = ald


def _final_body(s_ref, bias_ref, bnw_ref, bnb_ref, b_ref,
                w1_ref, b1_ref, w2_ref, b2_ref, o_ref, accs, accm):
    i = pl.program_id(0)

    @pl.when(i == 0)
    def _():
        accs[...] = jnp.zeros_like(accs)
        accm[...] = jnp.full_like(accm, -1e30)

    s = s_ref[...]
    num = s[:, :_G2]
    den = s[:, _G2:_G2 + 1]
    out = num / (den + 1e-16) + bias_ref[...]
    hh = out * (bnw_ref[...] * _INV) + bnb_ref[...]
    hact = jnp.where(hh > 0, hh, jnp.exp(hh) - 1.0)

    b = b_ref[0, 0, :]
    rid = lax.broadcasted_iota(jnp.int32, (_R, 1), 0) + i * _R
    valid = rid < _N
    ohT = ((b[None, :] ==
            lax.broadcasted_iota(jnp.int32, (_NUM_GRAPHS, _R), 0))
           & (valid[:, 0])[None, :]).astype(jnp.float32)
    hcnt = jnp.concatenate(
        [hact, jnp.ones((_R, 1), jnp.float32),
         jnp.zeros((_R, 2 * _G2 - _G2 - 1), jnp.float32)], axis=1)
    accs[...] += _HP(ohT, hcnt)

    for g in range(_NUM_GRAPHS):
        m = (b[:, None] == g) & valid
        hm = jnp.where(m, hact, -1e30)
        accm[g, :] = jnp.maximum(accm[g, :], jnp.max(hm, axis=0))

    @pl.when(i == _NBLK - 1)
    def _():
        sums = accs[:, :_G2]
        cnt = accs[:, _G2:_G2 + 1]
        hmean = sums / jnp.maximum(cnt, 1.0)
        gfeat = jnp.concatenate([hmean, accm[...]], axis=1)
        hid = jnp.maximum(_HP(gfeat, w1_ref[...]) + b1_ref[...], 0.0)
        o_ref[...] = _HP(hid, w2_ref[...]) + b2_ref[...]


def _full(shape):
    return pl.BlockSpec(shape, lambda i: (0,) * len(shape))


def _k0(x3, dep3, emb, dpw, dpb, lin, As, Ad):
    return pl.pallas_call(
        _k0_body,
        grid=(_NBLK,),
        in_specs=[
            pl.BlockSpec((1, 1, _R), lambda i: (i, 0, 0)),
            pl.BlockSpec((1, 1, _R), lambda i: (i, 0, 0)),
            _full(emb.shape), _full(dpw.shape), _full(dpb.shape),
            _full(lin.shape), _full(As.shape), _full(Ad.shape),
        ],
        out_specs=[
            pl.BlockSpec((_R, _TW0), lambda i: (i, 0)),
            pl.BlockSpec((_R, 4), lambda i: (i, 0)),
        ],
        out_shape=[
            jax.ShapeDtypeStruct((_NPAD, _TW0), jnp.float32),
            jax.ShapeDtypeStruct((_NPAD, 4), jnp.float32),
        ],
    )(x3, dep3, emb, dpw, dpb, lin, As, Ad)


def _kmid(S, bias, bnw, bnb, lin, As, Ad, Dp, Hp, ROWn):
    body = functools.partial(_mid_body, Dp=Dp, Hp=Hp, ROWn=ROWn)
    return pl.pallas_call(
        body,
        grid=(_NBLK,),
        in_specs=[
            pl.BlockSpec((_R, S.shape[1]), lambda i: (i, 0)),
            _full(bias.shape), _full(bnw.shape), _full(bnb.shape),
            _full(lin.shape), _full(As.shape), _full(Ad.shape),
        ],
        out_specs=[
            pl.BlockSpec((_R, ROWn), lambda i: (i, 0)),
            pl.BlockSpec((_R, 4), lambda i: (i, 0)),
        ],
        out_shape=[
            jax.ShapeDtypeStruct((_NPAD, ROWn), jnp.float32),
            jax.ShapeDtypeStruct((_NPAD, 4), jnp.float32),
        ],
    )(S, bias, bnw, bnb, lin, As, Ad)


def _kfinal(S, bias, bnw, bnb, b3, w1, b1, w2, b2):
    return pl.pallas_call(
        _final_body,
        grid=(_NBLK,),
        in_specs=[
            pl.BlockSpec((_R, _ROW2), lambda i: (i, 0)),
            _full(bias.shape), _full(bnw.shape), _full(bnb.shape),
            pl.BlockSpec((1, 1, _R), lambda i: (i, 0, 0)),
            _full(w1.shape), _full(b1.shape), _full(w2.shape),
            _full(b2.shape),
        ],
        out_specs=pl.BlockSpec((_NUM_GRAPHS, _NUM_CLASSES), lambda i: (0, 0)),
        out_shape=jax.ShapeDtypeStruct((_NUM_GRAPHS, _NUM_CLASSES),
                                       jnp.float32),
        scratch_shapes=[
            pltpu.VMEM((_NUM_GRAPHS, 2 * _G2), jnp.float32),
            pltpu.VMEM((_NUM_GRAPHS, _G2), jnp.float32),
        ],
    )(S, bias, bnw, bnb, b3, w1, b1, w2, b2)


def _blockdiag(a, D):
    H, DH = a.shape
    cols = []
    for h in range(H):
        col = jnp.zeros((D, 1), jnp.float32)
        col = lax.dynamic_update_slice(col, a[h][:, None], (h * DH, 0))
        cols.append(col)
    return jnp.concatenate(cols, axis=1)


def _pad_reshape_3d(v):
    v = jnp.concatenate([v, jnp.zeros((_NPAD - _N,), v.dtype)])
    return v.reshape(_NBLK, 1, _R)


def kernel(x, edge_index, depth, batch, emb, dpW, dpB, lin0, asrc0, adst0, bias0, bnw0, bnb0, lin1, asrc1, adst1, bias1, bnw1, bnb1, lin2, asrc2, adst2, bias2, bnw2, bnb2, cW1, cb1, cW2, cb2):
    loop = jnp.arange(_N, dtype=jnp.int32)
    src = jnp.concatenate([edge_index[0].astype(jnp.int32), loop])
    dst = jnp.concatenate([edge_index[1].astype(jnp.int32), loop])
    dsts, srcs = lax.sort((dst, src), num_keys=1)
    zpad = jnp.zeros((_EPAD - _EN,), jnp.int32)
    srcs = jnp.concatenate([srcs, zpad])
    dsts = jnp.concatenate([dsts, zpad])
    cuts = jnp.arange(225, dtype=jnp.int32) * 224
    lob = jnp.searchsorted(dsts[:_EN], cuts[:224]).astype(jnp.int32)
    bounds = jnp.concatenate(
        [lob, jnp.array([_EN], jnp.int32), jnp.zeros((15,), jnp.int32)])

    x3 = _pad_reshape_3d(x.astype(jnp.int32))
    dep3 = _pad_reshape_3d(depth)
    b3 = _pad_reshape_3d(batch.astype(jnp.int32))

    r1 = lambda v: v.reshape(1, -1)
    T0, ALD0 = _k0(x3, dep3, emb, dpW, r1(dpB), lin0,
                   _blockdiag(asrc0, _HEADS * _HID),
                   _blockdiag(adst0, _HEADS * _HID))
    S0 = _edge_kernel_l01(T0, ALD0.reshape(-1), srcs, dsts,
                          bounds).reshape(_NPAD, _ROW0)
    T1, ALD1 = _kmid(S0, r1(bias0), r1(bnw0), r1(bnb0), lin1,
                     _blockdiag(asrc1, _HEADS * _HID),
                     _blockdiag(adst1, _HEADS * _HID),
                     _HEADS * _HID, _HEADS, _TW0)
    S1 = _edge_kernel_l01(T1, ALD1.reshape(-1), srcs, dsts,
                          bounds).reshape(_NPAD, _ROW0)
    T2, ALD2 = _kmid(S1, r1(bias1), r1(bnw1), r1(bnb1), lin2,
                     _blockdiag(asrc2, _G2), _blockdiag(adst2, _G2),
                     _HEADS * _HID, _HEADS, _TW2)
    S2 = _edge_kernel_l2(T2, ALD2.reshape(-1), srcs, dsts,
                          bounds).reshape(_NPAD, _ROW2)
    return _kfinal(S2, r1(bias2), r1(bnw2), r1(bnb2), b3,
                   cW1, r1(cb1), cW2, r1(cb2))


# speedup vs baseline: 37.6629x; 1.1584x over previous
import dataclasses
import functools

import jax
import jax.numpy as jnp
from jax import lax
from jax.experimental import pallas as pl
from jax.experimental.pallas import tpu as pltpu
from jax.experimental.pallas import tpu_sc as plsc

_N = 50000
_E = 800000
_EN = _E + _N
_NUM_TYPES = 512
_NUM_GRAPHS = 64
_D_EMB = 64
_HID = 32
_HEADS = 4
_G2 = 64
_NUM_CLASSES = 20

_R = 512
_NPAD = 50176
_NBLK = _NPAD // _R
_CS = _NPAD // 8
_K = 128
_EPAD = 860160
_INV = float(1.0 / (1.0 + 1e-5) ** 0.5)
_TW0 = 256
_TW2 = 128
_ROW0 = 144
_ROW2 = 80


def _make_edge_kernel(GROW, AROW, H, D):
    DH = D // H
    CR = 224
    NCH = 7

    mesh = plsc.VectorSubcoreMesh(
        core_axis_name="c", subcore_axis_name="s", num_cores=2, num_subcores=16
    )
    cp = pltpu.CompilerParams()
    if "needs_layout_passes" in pltpu.CompilerParams.__dataclass_fields__:
        cp = dataclasses.replace(cp, needs_layout_passes=False)

    @functools.partial(
        pl.kernel,
        out_type=jax.ShapeDtypeStruct((_NPAD * AROW,), jnp.float32),
        mesh=mesh,
        compiler_params=cp,
        scratch_types=[
            pltpu.VMEM((_K,), jnp.int32),
            pltpu.VMEM((_K,), jnp.int32),
            pltpu.VMEM((_K,), jnp.int32),
            pltpu.VMEM((_K,), jnp.int32),
            pltpu.VMEM((_K, GROW), jnp.float32),
            pltpu.VMEM((_K, GROW), jnp.float32),
            pltpu.VMEM((_K,), jnp.int32),
            pltpu.VMEM((CR * 4,), jnp.float32),
            pltpu.VMEM((240,), jnp.int32),
            pltpu.VMEM(((CR + 8) * AROW,), jnp.float32),
            pltpu.SemaphoreType.DMA,
            pltpu.SemaphoreType.DMA,
            pltpu.SemaphoreType.DMA,
            pltpu.SemaphoreType.DMA,
        ],
    )
    def edge_kernel(t_hbm, ald_hbm, src_hbm, dst_hbm, bounds_hbm, s_hbm,
                    srcb0, dstb0, srcb1, dstb1, rows0, rows1, dstloc,
                    aldv, bvec, acc, semi0, semi1, semg0, semg1):
        core = lax.axis_index("c")
        sid = lax.axis_index("s")
        w = core * 16 + sid
        pltpu.sync_copy(bounds_hbm, bvec)

        def bscal(idx):
            v = plsc.load_gather(bvec, [jnp.full((16,), idx, jnp.int32)])
            return jnp.max(v)

        zf = jnp.zeros((16,), jnp.float32)
        sets = ((srcb0, dstb0, rows0, semi0, semg0),
                (srcb1, dstb1, rows1, semi1, semg1))

        @pl.loop(0, NCH)
        def _(i):
            c = w * NCH + i
            cbase = c * CR
            lo = bscal(c)
            hi = bscal(c + 1)

            @pl.loop(0, (CR + 8) * AROW // 16)
            def _(r):
                acc[pl.ds(r * 16, 16)] = zf

            pltpu.sync_copy(ald_hbm.at[pl.ds(cbase * 4, CR * 4)], aldv)

            lo_al = (lo // 8) * 8
            nb = (hi - lo_al + (_K - 1)) // _K
            t2 = (nb + 1) // 2

            def fire_idx(s, b):
                sb, db, _, smi, _ = sets[s]
                base = lo_al + b * _K
                pltpu.async_copy(src_hbm.at[pl.ds(base, _K)], sb, smi)
                pltpu.async_copy(dst_hbm.at[pl.ds(base, _K)], db, smi)

            def wait_idx(s):
                sb, db, _, smi, _ = sets[s]
                pltpu.make_async_copy(
                    src_hbm.at[pl.ds(lo_al, _K)], sb, smi).wait()
                pltpu.make_async_copy(
                    dst_hbm.at[pl.ds(lo_al, _K)], db, smi).wait()

            def fire_g(s):
                sb, _, rw, _, smg = sets[s]
                pltpu.async_copy(t_hbm.at[sb], rw, smg)

            def wait_g(s):
                sb, _, rw, _, smg = sets[s]
                pltpu.make_async_copy(t_hbm.at[sb], rw, smg).wait()

            def compute(s, b):
                _, db, rw, _, _ = sets[s]
                base = lo_al + b * _K
                for g in range(_K // 16):
                    ids = lax.iota(jnp.int32, 16) + (g * 16)
                    gid = ids + base
                    mask = (gid >= lo) & (gid < hi)
                    dv = db[pl.ds(g * 16, 16)] - cbase
                    dvc = jnp.minimum(jnp.maximum(dv, 0), CR - 1)
                    dstloc[pl.ds(g * 16, 16)] = (
                        jnp.where(mask, dvc, CR) * AROW)
                    for h in range(H):
                        colv = jnp.full((16,), D + h, jnp.int32)
                        als = plsc.load_gather(rw, [ids, colv])
                        ald = plsc.load_gather(aldv, [dvc * 4 + h])
                        e = als + ald
                        e = jnp.where(e > 0, e, 0.2 * e)
                        ex = jnp.exp(e)
                        plsc.store_scatter(rw, [ids, colv], ex)

                def edge_body(j, dvo):
                    jn = jnp.minimum(j + 1, _K - 1)
                    dvn = jnp.max(plsc.load_gather(
                        dstloc, [jnp.full((16,), jn, jnp.int32)]))
                    jv = jnp.full((16,), j, jnp.int32)
                    for h in range(H):
                        bc = plsc.load_gather(
                            rw, [jv, jnp.full((16,), D + h, jnp.int32)])
                        for q in range(DH // 16):
                            co = h * DH + q * 16
                            plsc.addupdate(
                                acc.at[pl.ds(dvo + co, 16)],
                                rw[j, pl.ds(co, 16)] * bc)
                    plsc.addupdate(acc.at[pl.ds(dvo + D, 16)],
                                   rw[j, pl.ds(D, 16)])
                    return dvn

                dv0 = jnp.max(plsc.load_gather(
                    dstloc, [jnp.zeros((16,), jnp.int32)]))
                lax.fori_loop(0, _K, edge_body, dv0, unroll=False)

            fire_idx(0, 0)
            wait_idx(0)
            fire_g(0)
            fire_idx(1, 1)

            @pl.loop(0, t2)
            def _(t):
                b0 = 2 * t
                wait_idx(1)
                fire_g(1)
                wait_g(0)
                compute(0, b0)
                fire_idx(0, b0 + 2)
                wait_idx(0)
                fire_g(0)
                wait_g(1)
                compute(1, b0 + 1)
                fire_idx(1, b0 + 3)

            wait_g(0)
            wait_idx(1)

            pltpu.sync_copy(acc.at[pl.ds(0, CR * AROW)],
                            s_hbm.at[pl.ds(cbase * AROW, CR * AROW)])

    return edge_kernel


_edge_kernel_l01 = _make_edge_kernel(_TW0, _ROW0, _HEADS, _HEADS * _HID)
_edge_kernel_l2 = _make_edge_kernel(_TW2, _ROW2, 1, _G2)


_HP = functools.partial(jnp.dot, precision=lax.Precision.HIGHEST)


def _k0_body(x_ref, dep_ref, emb_ref, dpw_ref, dpb_ref, lin_ref,
             as_ref, ad_ref, t_ref, ald_ref):
    ids = x_ref[0, 0, :]
    oh = (ids[:, None] ==
          lax.broadcasted_iota(jnp.int32, (_R, _NUM_TYPES), 1)
          ).astype(jnp.float32)
    h = _HP(oh, emb_ref[...])
    dep = dep_ref[0, 0, :]
    h = h + dep[:, None] * dpw_ref[...] + dpb_ref[...]
    hw = _HP(h, lin_ref[...])
    als = _HP(hw, as_ref[...])
    ald = _HP(hw, ad_ref[...])
    pad = jnp.zeros((_R, _TW0 - als.shape[1] - hw.shape[1]), jnp.float32)
    t_ref[...] = jnp.concatenate([hw, als, pad], axis=1)
    if ald.shape[1] < 4:
        ald = jnp.concatenate(
            [ald, jnp.zeros((_R, 4 - ald.shape[1]), jnp.float32)], axis=1)
    ald_ref[...] = ald


def _mid_body(s_ref, bias_ref, bnw_ref, bnb_ref, lin_ref, as_ref, ad_ref,
              t_ref, ald_ref, *, Dp, Hp, ROWn):
    DHp = Dp // Hp
    s = s_ref[...]
    outs = []
    for h in range(Hp):
        num = s[:, h * DHp:(h + 1) * DHp]
        den = s[:, Dp + h:Dp + h + 1]
        outs.append(num / (den + 1e-16))
    out = jnp.concatenate(outs, axis=1) if Hp > 1 else outs[0]
    out = out + bias_ref[...]
    hh = out * (bnw_ref[...] * _INV) + bnb_ref[...]
    hact = jnp.where(hh > 0, hh, jnp.exp(hh) - 1.0)
    hw = _HP(hact, lin_ref[...])
    als = _HP(hw, as_ref[...])
    ald = _HP(hw, ad_ref[...])
    pad = jnp.zeros((_R, ROWn - als.shape[1] - hw.shape[1]), jnp.float32)
    t_ref[...] = jnp.concatenate([hw, als, pad], axis=1)
    if ald.shape[1] < 4:
        ald = jnp.concatenate(
            [ald, jnp.zeros((_R, 4 - ald.shape[1]), jnp.float32)], axis=1)
    ald_ref[...] = ald


def _final_body(s_ref, bias_ref, bnw_ref, bnb_ref, b_ref,
                w1_ref, b1_ref, w2_ref, b2_ref, o_ref, accs, accm):
    i = pl.program_id(0)

    @pl.when(i == 0)
    def _():
        accs[...] = jnp.zeros_like(accs)
        accm[...] = jnp.full_like(accm, -1e30)

    s = s_ref[...]
    num = s[:, :_G2]
    den = s[:, _G2:_G2 + 1]
    out = num / (den + 1e-16) + bias_ref[...]
    hh = out * (bnw_ref[...] * _INV) + bnb_ref[...]
    hact = jnp.where(hh > 0, hh, jnp.exp(hh) - 1.0)

    b = b_ref[0, 0, :]
    rid = lax.broadcasted_iota(jnp.int32, (_R, 1), 0) + i * _R
    valid = rid < _N
    ohT = ((b[None, :] ==
            lax.broadcasted_iota(jnp.int32, (_NUM_GRAPHS, _R), 0))
           & (valid[:, 0])[None, :]).astype(jnp.float32)
    hcnt = jnp.concatenate(
        [hact, jnp.ones((_R, 1), jnp.float32),
         jnp.zeros((_R, 2 * _G2 - _G2 - 1), jnp.float32)], axis=1)
    accs[...] += _HP(ohT, hcnt)

    for g in range(_NUM_GRAPHS):
        m = (b[:, None] == g) & valid
        hm = jnp.where(m, hact, -1e30)
        accm[g, :] = jnp.maximum(accm[g, :], jnp.max(hm, axis=0))

    @pl.when(i == _NBLK - 1)
    def _():
        sums = accs[:, :_G2]
        cnt = accs[:, _G2:_G2 + 1]
        hmean = sums / jnp.maximum(cnt, 1.0)
        gfeat = jnp.concatenate([hmean, accm[...]], axis=1)
        hid = jnp.maximum(_HP(gfeat, w1_ref[...]) + b1_ref[...], 0.0)
        o_ref[...] = _HP(hid, w2_ref[...]) + b2_ref[...]


def _full(shape):
    return pl.BlockSpec(shape, lambda i: (0,) * len(shape))


def _k0(x3, dep3, emb, dpw, dpb, lin, As, Ad):
    return pl.pallas_call(
        _k0_body,
        grid=(_NBLK,),
        in_specs=[
            pl.BlockSpec((1, 1, _R), lambda i: (i, 0, 0)),
            pl.BlockSpec((1, 1, _R), lambda i: (i, 0, 0)),
            _full(emb.shape), _full(dpw.shape), _full(dpb.shape),
            _full(lin.shape), _full(As.shape), _full(Ad.shape),
        ],
        out_specs=[
            pl.BlockSpec((_R, _TW0), lambda i: (i, 0)),
            pl.BlockSpec((_R, 4), lambda i: (i, 0)),
        ],
        out_shape=[
            jax.ShapeDtypeStruct((_NPAD, _TW0), jnp.float32),
            jax.ShapeDtypeStruct((_NPAD, 4), jnp.float32),
        ],
    )(x3, dep3, emb, dpw, dpb, lin, As, Ad)


def _kmid(S, bias, bnw, bnb, lin, As, Ad, Dp, Hp, ROWn):
    body = functools.partial(_mid_body, Dp=Dp, Hp=Hp, ROWn=ROWn)
    return pl.pallas_call(
        body,
        grid=(_NBLK,),
        in_specs=[
            pl.BlockSpec((_R, S.shape[1]), lambda i: (i, 0)),
            _full(bias.shape), _full(bnw.shape), _full(bnb.shape),
            _full(lin.shape), _full(As.shape), _full(Ad.shape),
        ],
        out_specs=[
            pl.BlockSpec((_R, ROWn), lambda i: (i, 0)),
            pl.BlockSpec((_R, 4), lambda i: (i, 0)),
        ],
        out_shape=[
            jax.ShapeDtypeStruct((_NPAD, ROWn), jnp.float32),
            jax.ShapeDtypeStruct((_NPAD, 4), jnp.float32),
        ],
    )(S, bias, bnw, bnb, lin, As, Ad)


def _kfinal(S, bias, bnw, bnb, b3, w1, b1, w2, b2):
    return pl.pallas_call(
        _final_body,
        grid=(_NBLK,),
        in_specs=[
            pl.BlockSpec((_R, _ROW2), lambda i: (i, 0)),
            _full(bias.shape), _full(bnw.shape), _full(bnb.shape),
            pl.BlockSpec((1, 1, _R), lambda i: (i, 0, 0)),
            _full(w1.shape), _full(b1.shape), _full(w2.shape),
            _full(b2.shape),
        ],
        out_specs=pl.BlockSpec((_NUM_GRAPHS, _NUM_CLASSES), lambda i: (0, 0)),
        out_shape=jax.ShapeDtypeStruct((_NUM_GRAPHS, _NUM_CLASSES),
                                       jnp.float32),
        scratch_shapes=[
            pltpu.VMEM((_NUM_GRAPHS, 2 * _G2), jnp.float32),
            pltpu.VMEM((_NUM_GRAPHS, _G2), jnp.float32),
        ],
    )(S, bias, bnw, bnb, b3, w1, b1, w2, b2)


def _blockdiag(a, D):
    H, DH = a.shape
    cols = []
    for h in range(H):
        col = jnp.zeros((D, 1), jnp.float32)
        col = lax.dynamic_update_slice(col, a[h][:, None], (h * DH, 0))
        cols.append(col)
    return jnp.concatenate(cols, axis=1)


def _pad_reshape_3d(v):
    v = jnp.concatenate([v, jnp.zeros((_NPAD - _N,), v.dtype)])
    return v.reshape(_NBLK, 1, _R)


def kernel(x, edge_index, depth, batch, emb, dpW, dpB, lin0, asrc0, adst0, bias0, bnw0, bnb0, lin1, asrc1, adst1, bias1, bnw1, bnb1, lin2, asrc2, adst2, bias2, bnw2, bnb2, cW1, cb1, cW2, cb2):
    loop = jnp.arange(_N, dtype=jnp.int32)
    src = jnp.concatenate([edge_index[0].astype(jnp.int32), loop])
    dst = jnp.concatenate([edge_index[1].astype(jnp.int32), loop])
    dsts, srcs = lax.sort((dst, src), num_keys=1)
    zpad = jnp.zeros((_EPAD - _EN,), jnp.int32)
    srcs = jnp.concatenate([srcs, zpad])
    dsts = jnp.concatenate([dsts, zpad])
    cuts = jnp.arange(225, dtype=jnp.int32) * 224
    lob = jnp.searchsorted(dsts[:_EN], cuts[:224]).astype(jnp.int32)
    bounds = jnp.concatenate(
        [lob, jnp.array([_EN], jnp.int32), jnp.zeros((15,), jnp.int32)])

    x3 = _pad_reshape_3d(x.astype(jnp.int32))
    dep3 = _pad_reshape_3d(depth)
    b3 = _pad_reshape_3d(batch.astype(jnp.int32))

    r1 = lambda v: v.reshape(1, -1)
    T0, ALD0 = _k0(x3, dep3, emb, dpW, r1(dpB), lin0,
                   _blockdiag(asrc0, _HEADS * _HID),
                   _blockdiag(adst0, _HEADS * _HID))
    S0 = _edge_kernel_l01(T0, ALD0.reshape(-1), srcs, dsts,
                          bounds).reshape(_NPAD, _ROW0)
    T1, ALD1 = _kmid(S0, r1(bias0), r1(bnw0), r1(bnb0), lin1,
                     _blockdiag(asrc1, _HEADS * _HID),
                     _blockdiag(adst1, _HEADS * _HID),
                     _HEADS * _HID, _HEADS, _TW0)
    S1 = _edge_kernel_l01(T1, ALD1.reshape(-1), srcs, dsts,
                          bounds).reshape(_NPAD, _ROW0)
    T2, ALD2 = _kmid(S1, r1(bias1), r1(bnw1), r1(bnb1), lin2,
                     _blockdiag(asrc2, _G2), _blockdiag(adst2, _G2),
                     _HEADS * _HID, _HEADS, _TW2)
    S2 = _edge_kernel_l2(T2, ALD2.reshape(-1), srcs, dsts,
                          bounds).reshape(_NPAD, _ROW2)
    return _kfinal(S2, r1(bias2), r1(bnw2), r1(bnb2), b3,
                   cW1, r1(cb1), cW2, r1(cb2))
